# baseline (speedup 1.0000x reference)
import jax
import jax.numpy as jnp
from jax import lax
from jax.experimental import pallas as pl
from jax.experimental.pallas import tpu as pltpu
from jax.experimental.pallas import tpu_sc as plsc
from jax._src.pallas.mpmd import _mpmd_map

_CAP = 16384
_B = 4096
_D = 3 * 32 * 32

_NC = 2
_NS = 16
_NW = _NC * _NS
_RPW = _CAP // _NW
_NV = _B // 16
_CH = 16
_LW = _RPW + 32


def _win_body(idx_hbm, srcl_hbm, dstl_hbm, nwl_hbm,
              idx_v, last_v, srcf, dstf, sem_unused):
    del sem_unused
    wid = lax.axis_index("s") * _NC + lax.axis_index("c")
    base = wid * _RPW
    lane = lax.iota(jnp.int32, 16)

    pltpu.sync_copy(idx_hbm, idx_v)

    for s in range(_RPW // 16):
        last_v[pl.ds(s * 16, 16)] = jnp.full((16,), -1, jnp.int32)

    def win_step(v, _):
        iv = idx_v[pl.ds(v * 16, 16)]
        inr = (iv >= base) & (iv < base + _RPW)
        n_inr = jnp.sum(inr.astype(jnp.int32))

        @pl.when(n_inr > 0)
        def _():
            _, keep = plsc.scan_count(iv, mask=inr)
            plsc.store_scatter(last_v, [iv - base],
                               v * 16 + lane, mask=keep)
        return 0

    lax.fori_loop(0, _NV, win_step, 0)

    def cmp_step(s, off):
        lv = last_v[pl.ds(s * 16, 16)]
        m = lv >= 0
        cnt = jnp.sum(m.astype(jnp.int32))
        plsc.store_compressed(srcf.at[pl.ds(off, 16)], lv, mask=m)
        plsc.store_compressed(dstf.at[pl.ds(off, 16)], base + s * 16 + lane, mask=m)
        return off + cnt

    nw = lax.fori_loop(0, _RPW // 16, cmp_step, jnp.int32(0))

    @pl.when(nw > 0)
    def _():
        neg = jnp.full((16,), -(2**31), jnp.int32)
        zero16 = jnp.zeros((16,), jnp.int32)
        s0 = jnp.max(jnp.where(lane == 0, srcf[pl.ds(0, 16)], neg))
        d0 = jnp.max(jnp.where(lane == 0, dstf[pl.ds(0, 16)], neg))
        srcf[pl.ds(nw, 16)] = zero16 + s0
        dstf[pl.ds(nw, 16)] = zero16 + d0

    pltpu.sync_copy(srcf, srcl_hbm.at[pl.ds(wid * _LW, _LW)])
    pltpu.sync_copy(dstf, dstl_hbm.at[pl.ds(wid * _LW, _LW)])
    last_v[pl.ds(0, 16)] = jnp.zeros((16,), jnp.int32) + nw
    pltpu.sync_copy(last_v.at[pl.ds(0, 16)], nwl_hbm.at[pl.ds(wid * 16, 16)])


_win_lists = _mpmd_map(
    [(plsc.VectorSubcoreMesh(core_axis_name="c", subcore_axis_name="s"),
      _win_body)],
    [
        jax.ShapeDtypeStruct((_NW * _LW,), jnp.int32),
        jax.ShapeDtypeStruct((_NW * _LW,), jnp.int32),
        jax.ShapeDtypeStruct((_NW * 16,), jnp.int32),
    ],
    input_output_aliases={},
    scratch_types=[
        pltpu.VMEM((_B,), jnp.int32),
        pltpu.VMEM((_RPW,), jnp.int32),
        pltpu.VMEM((_LW,), jnp.int32),
        pltpu.VMEM((_LW,), jnp.int32),
        pltpu.SemaphoreType.DMA,
    ],
    compiler_params=pltpu.CompilerParams(needs_layout_passes=False),
)


def _mov_body(bx_hbm, by_hbm, bt_hbm, x_hbm, y_hbm, t_hbm,
              srcl_hbm, dstl_hbm, nwl_hbm,
              obx_hbm, oby_hbm, obt_hbm,
              srcf, dstf, nw_v, y_v, t_v, by_v, bt_v,
              buf_a, buf_b, sem_g, sem_s):
    del bx_hbm
    wid = lax.axis_index("s") * _NC + lax.axis_index("c")
    base = wid * _RPW

    pltpu.sync_copy(srcl_hbm.at[pl.ds(wid * _LW, _LW)], srcf)
    pltpu.sync_copy(dstl_hbm.at[pl.ds(wid * _LW, _LW)], dstf)
    pltpu.sync_copy(nwl_hbm.at[pl.ds(wid * 16, 16)], nw_v)
    pltpu.sync_copy(y_hbm, y_v)
    pltpu.sync_copy(t_hbm, t_v)
    pltpu.sync_copy(by_hbm.at[pl.ds(base, _RPW)], by_v)
    pltpu.sync_copy(bt_hbm.at[pl.ds(base, _RPW)], bt_v)
    nw = jnp.max(nw_v[pl.ds(0, 16)])
    nch = (nw + _CH - 1) // _CH

    def mov_pair(p, _):
        k0 = p * 2
        sv0 = srcf[pl.ds(k0 * _CH, _CH)]
        pltpu.make_async_copy(x_hbm.at[sv0], buf_a, sem_g).start()

        @pl.when(k0 + 1 < nch)
        def _():
            sv1 = srcf[pl.ds((k0 + 1) * _CH, _CH)]
            pltpu.make_async_copy(x_hbm.at[sv1], buf_b, sem_g).start()

        pltpu.make_async_copy(x_hbm.at[sv0], buf_a, sem_g).wait()
        dv0 = dstf[pl.ds(k0 * _CH, _CH)]
        pltpu.async_copy(buf_a, obx_hbm.at[dv0], sem_s).wait()

        @pl.when(k0 + 1 < nch)
        def _():
            sv1 = srcf[pl.ds((k0 + 1) * _CH, _CH)]
            pltpu.make_async_copy(x_hbm.at[sv1], buf_b, sem_g).wait()
            dv1 = dstf[pl.ds((k0 + 1) * _CH, _CH)]
            pltpu.async_copy(buf_b, obx_hbm.at[dv1], sem_s).wait()
        return 0

    lax.fori_loop(0, (nch + 1) // 2, mov_pair, 0)

    def lbl_step(w, _):
        sv = srcf[pl.ds(w * 16, 16)]
        dv = dstf[pl.ds(w * 16, 16)] - base
        plsc.store_scatter(by_v, [dv],
                           plsc.load_gather(y_v, [sv]))
        plsc.store_scatter(bt_v, [dv],
                           plsc.load_gather(t_v, [sv]))
        return 0

    lax.fori_loop(0, (nw + 15) // 16, lbl_step, 0)
    pltpu.sync_copy(by_v, oby_hbm.at[pl.ds(base, _RPW)])
    pltpu.sync_copy(bt_v, obt_hbm.at[pl.ds(base, _RPW)])


_sc_scatter = _mpmd_map(
    [(plsc.VectorSubcoreMesh(core_axis_name="c", subcore_axis_name="s"),
      _mov_body)],
    [
        jax.ShapeDtypeStruct((_CAP, _D), jnp.float32),
        jax.ShapeDtypeStruct((_CAP,), jnp.int32),
        jax.ShapeDtypeStruct((_CAP,), jnp.int32),
    ],
    input_output_aliases={0: 0, 1: 1, 2: 2},
    scratch_types=[
        pltpu.VMEM((_LW,), jnp.int32),
        pltpu.VMEM((_LW,), jnp.int32),
        pltpu.VMEM((16,), jnp.int32),
        pltpu.VMEM((_B,), jnp.int32),
        pltpu.VMEM((_B,), jnp.int32),
        pltpu.VMEM((_RPW,), jnp.int32),
        pltpu.VMEM((_RPW,), jnp.int32),
        pltpu.VMEM((_CH, _D), jnp.float32),
        pltpu.VMEM((_CH, _D), jnp.float32),
        pltpu.SemaphoreType.DMA,
        pltpu.SemaphoreType.DMA,
    ],
    compiler_params=pltpu.CompilerParams(needs_layout_passes=False),
)


_CBLK = 1024


def _copy_body(src_ref, dst_ref):
    dst_ref[...] = src_ref[...]


def _tc_copy(bx2):
    return pl.pallas_call(
        _copy_body,
        grid=(_CAP // _CBLK,),
        in_specs=[pl.BlockSpec((_CBLK, _D), lambda i: (i, 0))],
        out_specs=pl.BlockSpec((_CBLK, _D), lambda i: (i, 0)),
        out_shape=jax.ShapeDtypeStruct((_CAP, _D), jnp.float32),
    )(bx2)


def kernel(bx, by, bt, x, y, t, idx):
    srcl, dstl, nwl = _win_lists(idx)
    out1 = _tc_copy(bx.reshape(_CAP, _D))
    obx, oby, obt = _sc_scatter(
        out1, by, bt, x.reshape(_B, _D), y, t,
        srcl, dstl, nwl)
    return (obx.reshape(_CAP, 3, 32, 32), oby, obt)


# speedup vs baseline: 1.2756x; 1.2756x over previous
import jax
import jax.numpy as jnp
from jax import lax
from jax.experimental import pallas as pl
from jax.experimental.pallas import tpu as pltpu
from jax.experimental.pallas import tpu_sc as plsc
from jax._src.pallas.mpmd import _mpmd_map

_CAP = 16384
_B = 4096
_D = 3 * 32 * 32

_NC = 2
_NS = 16
_NW = _NC * _NS
_RPW = _CAP // _NW
_NV = _B // 16
_CH = 16
_LW = _RPW + 32


def _win_body(idx_hbm, srcl_hbm, dstl_hbm, nwl_hbm,
              idx_v, last_v, srcf, dstf, sem_unused):
    del sem_unused
    wid = lax.axis_index("s") * _NC + lax.axis_index("c")
    base = wid * _RPW
    lane = lax.iota(jnp.int32, 16)

    pltpu.sync_copy(idx_hbm, idx_v)

    for s in range(_RPW // 16):
        last_v[pl.ds(s * 16, 16)] = jnp.full((16,), -1, jnp.int32)

    def win_step(v, _):
        iv = idx_v[pl.ds(v * 16, 16)]
        inr = (iv >= base) & (iv < base + _RPW)
        n_inr = jnp.sum(inr.astype(jnp.int32))

        @pl.when(n_inr > 0)
        def _():
            _, keep = plsc.scan_count(iv, mask=inr)
            plsc.store_scatter(last_v, [iv - base],
                               v * 16 + lane, mask=keep)
        return 0

    lax.fori_loop(0, _NV, win_step, 0)

    def cmp_step(s, off):
        lv = last_v[pl.ds(s * 16, 16)]
        m = lv >= 0
        cnt = jnp.sum(m.astype(jnp.int32))
        plsc.store_compressed(srcf.at[pl.ds(off, 16)], lv, mask=m)
        plsc.store_compressed(dstf.at[pl.ds(off, 16)], base + s * 16 + lane, mask=m)
        return off + cnt

    nw = lax.fori_loop(0, _RPW // 16, cmp_step, jnp.int32(0))

    @pl.when(nw > 0)
    def _():
        neg = jnp.full((16,), -(2**31), jnp.int32)
        zero16 = jnp.zeros((16,), jnp.int32)
        s0 = jnp.max(jnp.where(lane == 0, srcf[pl.ds(0, 16)], neg))
        d0 = jnp.max(jnp.where(lane == 0, dstf[pl.ds(0, 16)], neg))
        srcf[pl.ds(nw, 16)] = zero16 + s0
        dstf[pl.ds(nw, 16)] = zero16 + d0

    pltpu.sync_copy(srcf, srcl_hbm.at[pl.ds(wid * _LW, _LW)])
    pltpu.sync_copy(dstf, dstl_hbm.at[pl.ds(wid * _LW, _LW)])
    last_v[pl.ds(0, 16)] = jnp.zeros((16,), jnp.int32) + nw
    pltpu.sync_copy(last_v.at[pl.ds(0, 16)], nwl_hbm.at[pl.ds(wid * 16, 16)])


_win_lists = _mpmd_map(
    [(plsc.VectorSubcoreMesh(core_axis_name="c", subcore_axis_name="s"),
      _win_body)],
    [
        jax.ShapeDtypeStruct((_NW * _LW,), jnp.int32),
        jax.ShapeDtypeStruct((_NW * _LW,), jnp.int32),
        jax.ShapeDtypeStruct((_NW * 16,), jnp.int32),
    ],
    input_output_aliases={},
    scratch_types=[
        pltpu.VMEM((_B,), jnp.int32),
        pltpu.VMEM((_RPW,), jnp.int32),
        pltpu.VMEM((_LW,), jnp.int32),
        pltpu.VMEM((_LW,), jnp.int32),
        pltpu.SemaphoreType.DMA,
    ],
    compiler_params=pltpu.CompilerParams(needs_layout_passes=False),
)


def _mov_body(bx_hbm, by_hbm, bt_hbm, x_hbm, y_hbm, t_hbm,
              srcl_hbm, dstl_hbm, nwl_hbm,
              obx_hbm, oby_hbm, obt_hbm,
              srcf, dstf, nw_v, y_v, t_v, by_v, bt_v,
              buf_a, buf_b, sem_g, sem_s):
    del bx_hbm
    wid = lax.axis_index("s") * _NC + lax.axis_index("c")
    base = wid * _RPW

    pltpu.sync_copy(srcl_hbm.at[pl.ds(wid * _LW, _LW)], srcf)
    pltpu.sync_copy(dstl_hbm.at[pl.ds(wid * _LW, _LW)], dstf)
    pltpu.sync_copy(nwl_hbm.at[pl.ds(wid * 16, 16)], nw_v)
    pltpu.sync_copy(y_hbm, y_v)
    pltpu.sync_copy(t_hbm, t_v)
    pltpu.sync_copy(by_hbm.at[pl.ds(base, _RPW)], by_v)
    pltpu.sync_copy(bt_hbm.at[pl.ds(base, _RPW)], bt_v)
    nw = jnp.max(nw_v[pl.ds(0, 16)])
    nch = (nw + _CH - 1) // _CH

    def mov_pair(p, _):
        k0 = p * 2
        sv0 = srcf[pl.ds(k0 * _CH, _CH)]
        pltpu.make_async_copy(x_hbm.at[sv0], buf_a, sem_g).start()

        @pl.when(k0 + 1 < nch)
        def _():
            sv1 = srcf[pl.ds((k0 + 1) * _CH, _CH)]
            pltpu.make_async_copy(x_hbm.at[sv1], buf_b, sem_g).start()

        pltpu.make_async_copy(x_hbm.at[sv0], buf_a, sem_g).wait()
        dv0 = dstf[pl.ds(k0 * _CH, _CH)]
        pltpu.async_copy(buf_a, obx_hbm.at[dv0], sem_s).wait()

        @pl.when(k0 + 1 < nch)
        def _():
            sv1 = srcf[pl.ds((k0 + 1) * _CH, _CH)]
            pltpu.make_async_copy(x_hbm.at[sv1], buf_b, sem_g).wait()
            dv1 = dstf[pl.ds((k0 + 1) * _CH, _CH)]
            pltpu.async_copy(buf_b, obx_hbm.at[dv1], sem_s).wait()
        return 0

    lax.fori_loop(0, (nch + 1) // 2, mov_pair, 0)

    def lbl_step(w, _):
        sv = srcf[pl.ds(w * 16, 16)]
        dv = dstf[pl.ds(w * 16, 16)] - base
        plsc.store_scatter(by_v, [dv],
                           plsc.load_gather(y_v, [sv]))
        plsc.store_scatter(bt_v, [dv],
                           plsc.load_gather(t_v, [sv]))
        return 0

    lax.fori_loop(0, (nw + 15) // 16, lbl_step, 0)
    pltpu.sync_copy(by_v, oby_hbm.at[pl.ds(base, _RPW)])
    pltpu.sync_copy(bt_v, obt_hbm.at[pl.ds(base, _RPW)])


_sc_scatter = _mpmd_map(
    [(plsc.VectorSubcoreMesh(core_axis_name="c", subcore_axis_name="s"),
      _mov_body)],
    [
        jax.ShapeDtypeStruct((_CAP, _D), jnp.float32),
        jax.ShapeDtypeStruct((_CAP,), jnp.int32),
        jax.ShapeDtypeStruct((_CAP,), jnp.int32),
    ],
    input_output_aliases={0: 0, 1: 1, 2: 2},
    scratch_types=[
        pltpu.VMEM((_LW,), jnp.int32),
        pltpu.VMEM((_LW,), jnp.int32),
        pltpu.VMEM((16,), jnp.int32),
        pltpu.VMEM((_B,), jnp.int32),
        pltpu.VMEM((_B,), jnp.int32),
        pltpu.VMEM((_RPW,), jnp.int32),
        pltpu.VMEM((_RPW,), jnp.int32),
        pltpu.VMEM((_CH, _D), jnp.float32),
        pltpu.VMEM((_CH, _D), jnp.float32),
        pltpu.SemaphoreType.DMA,
        pltpu.SemaphoreType.DMA,
    ],
    compiler_params=pltpu.CompilerParams(needs_layout_passes=False),
)


def kernel(bx, by, bt, x, y, t, idx):
    srcl, dstl, nwl = _win_lists(idx)
    obx, oby, obt = _sc_scatter(
        bx.reshape(_CAP, _D), by, bt, x.reshape(_B, _D), y, t,
        srcl, dstl, nwl)
    return (obx.reshape(_CAP, 3, 32, 32), oby, obt)


# speedup vs baseline: 1.2791x; 1.0028x over previous
import jax
import jax.numpy as jnp
from jax import lax
from jax.experimental import pallas as pl
from jax.experimental.pallas import tpu as pltpu
from jax.experimental.pallas import tpu_sc as plsc
from jax._src.pallas.mpmd import _mpmd_map

_CAP = 16384
_B = 4096
_D = 3 * 32 * 32

_NC = 2
_NS = 16
_NW = _NC * _NS
_RPW = _CAP // _NW
_NV = _B // 16
_CH = 16
_LW = _RPW + 32


def _win_body(idx_hbm, by_hbm, bt_hbm, y_hbm, t_hbm,
              srcl_hbm, dstl_hbm, nwl_hbm, oby_hbm, obt_hbm,
              idx_v, last_v, srcf, dstf, y_v, t_v, by_v, bt_v, sem_unused):
    del sem_unused
    wid = lax.axis_index("s") * _NC + lax.axis_index("c")
    base = wid * _RPW
    lane = lax.iota(jnp.int32, 16)

    pltpu.sync_copy(idx_hbm, idx_v)
    pltpu.sync_copy(y_hbm, y_v)
    pltpu.sync_copy(t_hbm, t_v)
    pltpu.sync_copy(by_hbm.at[pl.ds(base, _RPW)], by_v)
    pltpu.sync_copy(bt_hbm.at[pl.ds(base, _RPW)], bt_v)

    for s in range(_RPW // 16):
        last_v[pl.ds(s * 16, 16)] = jnp.full((16,), -1, jnp.int32)

    def win_step(v, _):
        iv = idx_v[pl.ds(v * 16, 16)]
        inr = (iv >= base) & (iv < base + _RPW)
        n_inr = jnp.sum(inr.astype(jnp.int32))

        @pl.when(n_inr > 0)
        def _():
            _, keep = plsc.scan_count(iv, mask=inr)
            plsc.store_scatter(last_v, [iv - base],
                               v * 16 + lane, mask=keep)
        return 0

    lax.fori_loop(0, _NV, win_step, 0)

    def cmp_step(s, off):
        lv = last_v[pl.ds(s * 16, 16)]
        m = lv >= 0
        cnt = jnp.sum(m.astype(jnp.int32))
        plsc.store_compressed(srcf.at[pl.ds(off, 16)], lv, mask=m)
        plsc.store_compressed(dstf.at[pl.ds(off, 16)], base + s * 16 + lane, mask=m)
        return off + cnt

    nw = lax.fori_loop(0, _RPW // 16, cmp_step, jnp.int32(0))

    @pl.when(nw > 0)
    def _():
        neg = jnp.full((16,), -(2**31), jnp.int32)
        zero16 = jnp.zeros((16,), jnp.int32)
        s0 = jnp.max(jnp.where(lane == 0, srcf[pl.ds(0, 16)], neg))
        d0 = jnp.max(jnp.where(lane == 0, dstf[pl.ds(0, 16)], neg))
        srcf[pl.ds(nw, 16)] = zero16 + s0
        dstf[pl.ds(nw, 16)] = zero16 + d0

    pltpu.sync_copy(srcf, srcl_hbm.at[pl.ds(wid * _LW, _LW)])
    pltpu.sync_copy(dstf, dstl_hbm.at[pl.ds(wid * _LW, _LW)])
    last_v[pl.ds(0, 16)] = jnp.zeros((16,), jnp.int32) + nw
    pltpu.sync_copy(last_v.at[pl.ds(0, 16)], nwl_hbm.at[pl.ds(wid * 16, 16)])

    def lbl_step(w, _):
        sv = srcf[pl.ds(w * 16, 16)]
        dv = dstf[pl.ds(w * 16, 16)] - base
        plsc.store_scatter(by_v, [dv],
                           plsc.load_gather(y_v, [sv]))
        plsc.store_scatter(bt_v, [dv],
                           plsc.load_gather(t_v, [sv]))
        return 0

    lax.fori_loop(0, (nw + 15) // 16, lbl_step, 0)
    pltpu.sync_copy(by_v, oby_hbm.at[pl.ds(base, _RPW)])
    pltpu.sync_copy(bt_v, obt_hbm.at[pl.ds(base, _RPW)])


_win_lists = _mpmd_map(
    [(plsc.VectorSubcoreMesh(core_axis_name="c", subcore_axis_name="s"),
      _win_body)],
    [
        jax.ShapeDtypeStruct((_NW * _LW,), jnp.int32),
        jax.ShapeDtypeStruct((_NW * _LW,), jnp.int32),
        jax.ShapeDtypeStruct((_NW * 16,), jnp.int32),
        jax.ShapeDtypeStruct((_CAP,), jnp.int32),
        jax.ShapeDtypeStruct((_CAP,), jnp.int32),
    ],
    input_output_aliases={1: 3, 2: 4},
    scratch_types=[
        pltpu.VMEM((_B,), jnp.int32),
        pltpu.VMEM((_RPW,), jnp.int32),
        pltpu.VMEM((_LW,), jnp.int32),
        pltpu.VMEM((_LW,), jnp.int32),
        pltpu.VMEM((_B,), jnp.int32),
        pltpu.VMEM((_B,), jnp.int32),
        pltpu.VMEM((_RPW,), jnp.int32),
        pltpu.VMEM((_RPW,), jnp.int32),
        pltpu.SemaphoreType.DMA,
    ],
    compiler_params=pltpu.CompilerParams(needs_layout_passes=False),
)


def _mov_body(bx_hbm, x_hbm, srcl_hbm, dstl_hbm, nwl_hbm,
              obx_hbm,
              srcf, dstf, nw_v, buf_a, buf_b, sem_g, sem_s):
    del bx_hbm
    wid = lax.axis_index("s") * _NC + lax.axis_index("c")

    pltpu.sync_copy(srcl_hbm.at[pl.ds(wid * _LW, _LW)], srcf)
    pltpu.sync_copy(dstl_hbm.at[pl.ds(wid * _LW, _LW)], dstf)
    pltpu.sync_copy(nwl_hbm.at[pl.ds(wid * 16, 16)], nw_v)
    nw = jnp.max(nw_v[pl.ds(0, 16)])
    nch = (nw + _CH - 1) // _CH

    def mov_pair(p, _):
        k0 = p * 2
        sv0 = srcf[pl.ds(k0 * _CH, _CH)]
        pltpu.make_async_copy(x_hbm.at[sv0], buf_a, sem_g).start()

        @pl.when(k0 + 1 < nch)
        def _():
            sv1 = srcf[pl.ds((k0 + 1) * _CH, _CH)]
            pltpu.make_async_copy(x_hbm.at[sv1], buf_b, sem_g).start()

        pltpu.make_async_copy(x_hbm.at[sv0], buf_a, sem_g).wait()
        dv0 = dstf[pl.ds(k0 * _CH, _CH)]
        pltpu.async_copy(buf_a, obx_hbm.at[dv0], sem_s).wait()

        @pl.when(k0 + 1 < nch)
        def _():
            sv1 = srcf[pl.ds((k0 + 1) * _CH, _CH)]
            pltpu.make_async_copy(x_hbm.at[sv1], buf_b, sem_g).wait()
            dv1 = dstf[pl.ds((k0 + 1) * _CH, _CH)]
            pltpu.async_copy(buf_b, obx_hbm.at[dv1], sem_s).wait()
        return 0

    lax.fori_loop(0, (nch + 1) // 2, mov_pair, 0)


_sc_scatter = _mpmd_map(
    [(plsc.VectorSubcoreMesh(core_axis_name="c", subcore_axis_name="s"),
      _mov_body)],
    [
        jax.ShapeDtypeStruct((_CAP, _D), jnp.float32),
    ],
    input_output_aliases={0: 0},
    scratch_types=[
        pltpu.VMEM((_LW,), jnp.int32),
        pltpu.VMEM((_LW,), jnp.int32),
        pltpu.VMEM((16,), jnp.int32),
        pltpu.VMEM((_CH, _D), jnp.float32),
        pltpu.VMEM((_CH, _D), jnp.float32),
        pltpu.SemaphoreType.DMA,
        pltpu.SemaphoreType.DMA,
    ],
    compiler_params=pltpu.CompilerParams(needs_layout_passes=False),
)


def kernel(bx, by, bt, x, y, t, idx):
    srcl, dstl, nwl, oby, obt = _win_lists(idx, by, bt, y, t)
    (obx,) = _sc_scatter(bx.reshape(_CAP, _D), x.reshape(_B, _D),
                         srcl, dstl, nwl)
    return (obx.reshape(_CAP, 3, 32, 32), oby, obt)
